# baseline (speedup 1.0000x reference)
import functools

import jax
import jax.numpy as jnp
from jax import lax
from jax.experimental import pallas as pl
from jax.experimental.pallas import tpu as pltpu
from jax.experimental.pallas import tpu_sc as plsc

NC, NS = 2, 16
NW = NC * NS
L = 16
C = 128
NB = 4


G = 8


def _make_sc_msg(N_pad, n_per, H):
    rows_per = N_pad // NS
    n_zero = rows_per // C
    HJ = H // L
    n_grp = n_per // G
    assert n_per % G == 0 and n_grp % 2 == 0 and G % 2 == 0
    assert rows_per % C == 0

    mesh = plsc.VectorSubcoreMesh(core_axis_name="c", subcore_axis_name="s")

    @functools.partial(
        pl.kernel,
        out_type=jax.ShapeDtypeStruct((NC, N_pad, H), jnp.float32),
        mesh=mesh,
        compiler_params=pltpu.CompilerParams(needs_layout_passes=False),
        scratch_types=[
            pltpu.VMEM((H,), jnp.float32),
            pltpu.VMEM((2, G, C), jnp.int32),
            pltpu.VMEM((2, G, C), jnp.int32),
            pltpu.VMEM((2, G, C), jnp.float32),
            pltpu.VMEM_SHARED((N_pad, H), jnp.float32),
        ] + [pltpu.VMEM((C, H), jnp.float32)] * 2
          + [pltpu.SemaphoreType.DMA] * 4,
    )
    def sc_msg(p_hbm, src_hbm, dst_hbm, tr_hbm, weh_hbm, agg_hbm,
               weh_v, sidx_v, didx_v, tr_v, agg_sh,
               buf0, buf1, gsem0, gsem1, ssem0, ssem1):
        bufs = (buf0, buf1)
        gsem = (gsem0, gsem1)
        ssem = (ssem0, ssem1)
        c = lax.axis_index("c")
        s = lax.axis_index("s")
        wid = c * NS + s
        w0 = wid * n_per

        pltpu.sync_copy(weh_hbm, weh_v)

        def _stage(g, pg):
            row0 = w0 + g * G
            pltpu.async_copy(src_hbm.at[pl.ds(row0, G)], sidx_v.at[pg],
                             ssem[pg])
            pltpu.async_copy(dst_hbm.at[pl.ds(row0, G)], didx_v.at[pg],
                             ssem[pg])
            pltpu.async_copy(tr_hbm.at[pl.ds(row0, G)], tr_v.at[pg],
                             ssem[pg])

        def _drain_stage(g, pg):
            row0 = w0 + g * G
            pltpu.make_async_copy(src_hbm.at[pl.ds(row0, G)], sidx_v.at[pg],
                                  ssem[pg]).wait()
            pltpu.make_async_copy(dst_hbm.at[pl.ds(row0, G)], didx_v.at[pg],
                                  ssem[pg]).wait()
            pltpu.make_async_copy(tr_hbm.at[pl.ds(row0, G)], tr_v.at[pg],
                                  ssem[pg]).wait()

        _stage(0, 0)

        def _zrow(i, _):
            for j in range(HJ):
                buf0[i, pl.ds(j * L, L)] = jnp.zeros((L,), jnp.float32)
            return 0
        lax.fori_loop(0, C, _zrow, 0)
        for k in range(n_zero):
            pltpu.sync_copy(buf0, agg_sh.at[pl.ds(s * rows_per + k * C, C)])
        plsc.subcore_barrier()

        weh = [weh_v[pl.ds(j * L, L)] for j in range(HJ)]

        def _gather(pg, kl, b):
            pltpu.async_copy(p_hbm.at[sidx_v.at[pg, kl]], bufs[b], gsem[b])

        def _compute(pg, kl, b):
            buf = bufs[b]
            pltpu.make_async_copy(p_hbm.at[sidx_v.at[pg, kl]], buf,
                                  gsem[b]).wait()

            def _grp(g, _):
                trv = tr_v[pg, kl, pl.ds(g * L, L)]
                for i in range(L):
                    t = trv[i]
                    e = g * L + i
                    for j in range(HJ):
                        v = buf[e, pl.ds(j * L, L)]
                        buf[e, pl.ds(j * L, L)] = jnp.maximum(
                            v + t * weh[j], 0.0)
                return 0
            lax.fori_loop(0, C // L, _grp, 0)
            pltpu.sync_copy(buf, agg_sh.at[didx_v.at[pg, kl]], add=True)

        def _group(g, pg):
            @pl.when(g + 1 < n_grp)
            def _():
                _stage(g + 1, 1 - pg)
            _drain_stage(g, pg)
            _gather(pg, 0, 0)

            def _pairs(q, _):
                for b in range(2):
                    kl = q * 2 + b

                    @pl.when(kl + 1 < G)
                    def _():
                        _gather(pg, kl + 1, 1 - b)
                    _compute(pg, kl, b)
                return 0
            lax.fori_loop(0, G // 2, _pairs, 0)

        def _gpair(gq, _):
            _group(gq * 2, 0)
            _group(gq * 2 + 1, 1)
            return 0
        lax.fori_loop(0, n_grp // 2, _gpair, 0)

        plsc.subcore_barrier()
        pltpu.sync_copy(agg_sh.at[pl.ds(s * rows_per, rows_per)],
                        agg_hbm.at[c, pl.ds(s * rows_per, rows_per)])

    return sc_msg


def _make_sc_hint(N, n_per, H):
    HJ = H // L
    NBH = 2

    mesh = plsc.VectorSubcoreMesh(core_axis_name="c", subcore_axis_name="s")

    @functools.partial(
        pl.kernel,
        out_type=jax.ShapeDtypeStruct((NW * n_per, C), jnp.float32),
        mesh=mesh,
        compiler_params=pltpu.CompilerParams(needs_layout_passes=False),
        scratch_types=[
            pltpu.VMEM((n_per, C), jnp.int32),
            pltpu.VMEM((n_per, C), jnp.int32),
            pltpu.VMEM((n_per, C), jnp.float32),
        ] + [pltpu.VMEM((C, H), jnp.float32)] * (2 * NBH)
          + [pltpu.SemaphoreType.DMA] * (2 * NBH),
    )
    def sc_hint(h_hbm, hw_hbm, src_hbm, dst_hbm, pred_hbm,
                sidx_v, didx_v, pr_v, *rest):
        bufa = rest[:NBH]
        bufb = rest[NBH:2 * NBH]
        sema = rest[2 * NBH:3 * NBH]
        semb = rest[3 * NBH:]
        c = lax.axis_index("c")
        s = lax.axis_index("s")
        wid = c * NS + s
        w0 = wid * n_per

        pltpu.sync_copy(src_hbm.at[pl.ds(w0, n_per)], sidx_v)
        pltpu.sync_copy(dst_hbm.at[pl.ds(w0, n_per)], didx_v)

        def _gather(k, b):
            pltpu.async_copy(h_hbm.at[sidx_v.at[k]], bufa[b], sema[b])
            pltpu.async_copy(hw_hbm.at[didx_v.at[k]], bufb[b], semb[b])

        _gather(0, 0)
        lane = lax.iota(jnp.int32, L)

        def _pair(q, _):
            for b in range(NBH):
                k = q * NBH + b
                nb = (b + 1) % NBH

                @pl.when(k + 1 < n_per)
                def _():
                    _gather(k + 1, nb)

                pltpu.make_async_copy(h_hbm.at[sidx_v.at[k]], bufa[b],
                                      sema[b]).wait()
                pltpu.make_async_copy(hw_hbm.at[didx_v.at[k]], bufb[b],
                                      semb[b]).wait()
                ba, bb = bufa[b], bufb[b]

                def _grp(g, _):
                    vec = jnp.zeros((L,), jnp.float32)
                    for i in range(L):
                        e = g * L + i
                        acc = ba[e, pl.ds(0, L)] * bb[e, pl.ds(0, L)]
                        for j in range(1, HJ):
                            acc = acc + (ba[e, pl.ds(j * L, L)]
                                         * bb[e, pl.ds(j * L, L)])
                        vec = jnp.where(lane == i, jnp.sum(acc), vec)
                    pr_v[k, pl.ds(g * L, L)] = vec
                    return 0
                lax.fori_loop(0, C // L, _grp, 0)
            return 0
        lax.fori_loop(0, n_per // NBH, _pair, 0)

        pltpu.sync_copy(pr_v, pred_hbm.at[pl.ds(w0, n_per)])

    return sc_hint


def _make_tc_encode(N, D, H, R):
    row = pl.BlockSpec((R, D), lambda i: (i, 0))
    full = pl.BlockSpec((D, H), lambda i: (0, 0))
    vec = pl.BlockSpec((1, H), lambda i: (0, 0))
    out_row = pl.BlockSpec((R, H), lambda i: (i, 0))

    def body(x_ref, wenc_ref, wmsg_ref, b_ref, xe_ref, p_ref):
        xe = jnp.dot(x_ref[...], wenc_ref[...],
                     preferred_element_type=jnp.float32,
                     precision=lax.Precision.HIGHEST)
        xe_ref[...] = xe
        p_ref[...] = jnp.dot(xe, wmsg_ref[...],
                             preferred_element_type=jnp.float32,
                             precision=lax.Precision.HIGHEST) + b_ref[...]

    return pl.pallas_call(
        body,
        grid=(N // R,),
        in_specs=[row, full, full, vec],
        out_specs=[out_row, out_row],
        out_shape=[jax.ShapeDtypeStruct((N, H), jnp.float32)] * 2,
    )


def _make_tc_update(N, N_pad, H, R, with_p):
    row = pl.BlockSpec((R, H), lambda i: (i, 0))
    agg_spec = pl.BlockSpec((NC, R, H), lambda i: (0, i, 0))
    full = pl.BlockSpec((H, H), lambda i: (0, 0))
    vec = pl.BlockSpec((1, H), lambda i: (0, 0))

    def body(agg_ref, h_ref, xe_ref, wagg_ref, wself_ref, wmsg_ref, b_ref,
             whint_ref, hn_ref, hw_ref, *p_refs):
        agg = agg_ref[0] + agg_ref[1]
        hn = jnp.dot(agg, wagg_ref[...], preferred_element_type=jnp.float32,
                     precision=lax.Precision.HIGHEST)
        hn = hn + jnp.dot(h_ref[...], wself_ref[...],
                          preferred_element_type=jnp.float32,
                          precision=lax.Precision.HIGHEST)
        hn = jnp.maximum(hn, 0.0)
        hn_ref[...] = hn
        hw_ref[...] = hn * whint_ref[...]
        if with_p:
            p_refs[0][...] = (
                jnp.dot(hn + xe_ref[...], wmsg_ref[...],
                        preferred_element_type=jnp.float32,
                        precision=lax.Precision.HIGHEST) + b_ref[...])

    n_out = 3 if with_p else 2
    return pl.pallas_call(
        body,
        grid=(N // R,),
        in_specs=[agg_spec, row, row, full, full, full, vec, vec],
        out_specs=[row] * n_out,
        out_shape=[jax.ShapeDtypeStruct((N, H), jnp.float32)] * n_out,
    )


def _make_tc_out(N, H, R):
    row = pl.BlockSpec((R, H), lambda i: (i, 0))
    vec = pl.BlockSpec((1, H), lambda i: (0, 0))
    out_spec = pl.BlockSpec((R, 1), lambda i: (i, 0))

    def body(h_ref, wout_ref, o_ref):
        o_ref[...] = jnp.sum(h_ref[...] * wout_ref[...], axis=1, keepdims=True)

    return pl.pallas_call(
        body,
        grid=(N // R,),
        in_specs=[row, vec],
        out_specs=out_spec,
        out_shape=jax.ShapeDtypeStruct((N, 1), jnp.float32),
    )


def kernel(x, edge_index, trace_h, W_enc, W_msg, b_msg, w_eh, W_agg, W_self,
           w_hint, W_out):
    N, D = x.shape
    H = W_msg.shape[0]
    T, E = trace_h.shape
    R = 1000
    N_pad = -(-N // (NS * 128)) * (NS * 128)
    n_per = -(-E // (NW * C * 2 * G)) * (2 * G)
    E_pad = NW * n_per * C
    assert N % R == 0

    src = edge_index[0]
    dst = edge_index[1]
    pad = E_pad - E
    src_p = jnp.pad(src, (0, pad)).reshape(-1, C)
    dstm_p = jnp.pad(dst, (0, pad), constant_values=N_pad - 1).reshape(-1, C)
    dsth_p = jnp.pad(dst, (0, pad)).reshape(-1, C)
    tr_p = jnp.pad(trace_h, ((0, 0), (0, pad))).reshape(T, -1, C)
    b2 = b_msg.reshape(1, H)
    whint2 = w_hint.reshape(1, H)

    sc_msg = _make_sc_msg(N_pad, n_per, H)
    sc_hint = _make_sc_hint(N, n_per, H)
    tc_encode = _make_tc_encode(N, D, H, R)
    tc_update = _make_tc_update(N, N_pad, H, R, True)
    tc_update_last = _make_tc_update(N, N_pad, H, R, False)
    tc_out = _make_tc_out(N, H, R)

    xe, p = tc_encode(x, W_enc, W_msg, b2)
    h = jnp.zeros((N, H), dtype=x.dtype)
    preds = []
    for t in range(T):
        agg = sc_msg(p, src_p, dstm_p, tr_p[t], w_eh)
        if t + 1 < T:
            h, hw, p = tc_update(agg, h, xe, W_agg, W_self, W_msg, b2, whint2)
        else:
            h, hw = tc_update_last(agg, h, xe, W_agg, W_self, W_msg, b2,
                                   whint2)
        preds.append(sc_hint(h, hw, src_p, dsth_p).reshape(-1)[:E])
    out = tc_out(h, W_out.reshape(1, H))
    return out, jnp.stack(preds, axis=0)


# speedup vs baseline: 1.0168x; 1.0168x over previous
import functools

import jax
import jax.numpy as jnp
from jax import lax
from jax.experimental import pallas as pl
from jax.experimental.pallas import tpu as pltpu
from jax.experimental.pallas import tpu_sc as plsc

NC, NS = 2, 16
NW = NC * NS
L = 16
C = 128
NB = 4


G = 8


def _make_sc_msg(N_pad, n_per, H):
    rows_per = N_pad // NS
    n_zero = rows_per // C
    HJ = H // L
    n_grp = n_per // G
    assert n_per % G == 0 and n_grp % 2 == 0 and G % 2 == 0
    assert rows_per % C == 0

    mesh = plsc.VectorSubcoreMesh(core_axis_name="c", subcore_axis_name="s")

    @functools.partial(
        pl.kernel,
        out_type=jax.ShapeDtypeStruct((NC, N_pad, H), jnp.float32),
        mesh=mesh,
        compiler_params=pltpu.CompilerParams(needs_layout_passes=False),
        scratch_types=[
            pltpu.VMEM((H,), jnp.float32),
            pltpu.VMEM((2, G, C), jnp.int32),
            pltpu.VMEM((2, G, C), jnp.int32),
            pltpu.VMEM((2, G, C), jnp.float32),
            pltpu.VMEM_SHARED((N_pad, H), jnp.float32),
        ] + [pltpu.VMEM((C, H), jnp.float32)] * 2
          + [pltpu.SemaphoreType.DMA] * 6,
    )
    def sc_msg(p_hbm, src_hbm, dst_hbm, tr_hbm, weh_hbm, agg_hbm,
               weh_v, sidx_v, didx_v, tr_v, agg_sh,
               buf0, buf1, gsem0, gsem1, ssem0, ssem1, csem0, csem1):
        bufs = (buf0, buf1)
        gsem = (gsem0, gsem1)
        ssem = (ssem0, ssem1)
        csem = (csem0, csem1)
        c = lax.axis_index("c")
        s = lax.axis_index("s")
        wid = c * NS + s
        w0 = wid * n_per

        pltpu.sync_copy(weh_hbm, weh_v)

        def _stage(g, pg):
            row0 = w0 + g * G
            pltpu.async_copy(src_hbm.at[pl.ds(row0, G)], sidx_v.at[pg],
                             ssem[pg])
            pltpu.async_copy(dst_hbm.at[pl.ds(row0, G)], didx_v.at[pg],
                             ssem[pg])
            pltpu.async_copy(tr_hbm.at[pl.ds(row0, G)], tr_v.at[pg],
                             ssem[pg])

        def _drain_stage(g, pg):
            row0 = w0 + g * G
            pltpu.make_async_copy(src_hbm.at[pl.ds(row0, G)], sidx_v.at[pg],
                                  ssem[pg]).wait()
            pltpu.make_async_copy(dst_hbm.at[pl.ds(row0, G)], didx_v.at[pg],
                                  ssem[pg]).wait()
            pltpu.make_async_copy(tr_hbm.at[pl.ds(row0, G)], tr_v.at[pg],
                                  ssem[pg]).wait()

        _stage(0, 0)

        def _zrow(i, _):
            for j in range(HJ):
                buf0[i, pl.ds(j * L, L)] = jnp.zeros((L,), jnp.float32)
            return 0
        lax.fori_loop(0, C, _zrow, 0)
        for k in range(n_zero):
            pltpu.sync_copy(buf0, agg_sh.at[pl.ds(s * rows_per + k * C, C)])
        plsc.subcore_barrier()

        weh = [weh_v[pl.ds(j * L, L)] for j in range(HJ)]

        def _gather(pg, kl, b):
            pltpu.async_copy(p_hbm.at[sidx_v.at[pg, kl]], bufs[b], gsem[b])

        def _wait_scat(b, pg):
            pltpu.make_async_copy(bufs[b], agg_sh.at[didx_v.at[pg, 0]],
                                  csem[b]).wait()

        def _compute(pg, kl, b):
            buf = bufs[b]
            pltpu.make_async_copy(p_hbm.at[sidx_v.at[pg, kl]], buf,
                                  gsem[b]).wait()

            def _grp(g, _):
                trv = tr_v[pg, kl, pl.ds(g * L, L)]
                for i in range(L):
                    t = trv[i]
                    e = g * L + i
                    for j in range(HJ):
                        v = buf[e, pl.ds(j * L, L)]
                        buf[e, pl.ds(j * L, L)] = jnp.maximum(
                            v + t * weh[j], 0.0)
                return 0
            lax.fori_loop(0, C // L, _grp, 0)
            pltpu.async_copy(buf, agg_sh.at[didx_v.at[pg, kl]], csem[b],
                             add=True)

        def _group(g, pg):
            @pl.when(g + 1 < n_grp)
            def _():
                _stage(g + 1, 1 - pg)
            _drain_stage(g, pg)

            @pl.when(g > 0)
            def _():
                _wait_scat(0, pg)
            _gather(pg, 0, 0)

            def _pairs(q, _):
                for b in range(2):
                    kl = q * 2 + b

                    @pl.when(kl + 1 < G)
                    def _():
                        @pl.when((g > 0) | (kl > 0))
                        def _():
                            _wait_scat(1 - b, pg)
                        _gather(pg, kl + 1, 1 - b)
                    _compute(pg, kl, b)
                return 0
            lax.fori_loop(0, G // 2, _pairs, 0)

        def _gpair(gq, _):
            _group(gq * 2, 0)
            _group(gq * 2 + 1, 1)
            return 0
        lax.fori_loop(0, n_grp // 2, _gpair, 0)

        _wait_scat(0, 1)
        _wait_scat(1, 1)
        plsc.subcore_barrier()
        pltpu.sync_copy(agg_sh.at[pl.ds(s * rows_per, rows_per)],
                        agg_hbm.at[c, pl.ds(s * rows_per, rows_per)])

    return sc_msg


def _make_sc_hint(N, n_per, H):
    HJ = H // L
    NBH = 2

    mesh = plsc.VectorSubcoreMesh(core_axis_name="c", subcore_axis_name="s")

    @functools.partial(
        pl.kernel,
        out_type=jax.ShapeDtypeStruct((NW * n_per, C), jnp.float32),
        mesh=mesh,
        compiler_params=pltpu.CompilerParams(needs_layout_passes=False),
        scratch_types=[
            pltpu.VMEM((n_per, C), jnp.int32),
            pltpu.VMEM((n_per, C), jnp.int32),
            pltpu.VMEM((n_per, C), jnp.float32),
        ] + [pltpu.VMEM((C, H), jnp.float32)] * (2 * NBH)
          + [pltpu.SemaphoreType.DMA] * (2 * NBH),
    )
    def sc_hint(h_hbm, hw_hbm, src_hbm, dst_hbm, pred_hbm,
                sidx_v, didx_v, pr_v, *rest):
        bufa = rest[:NBH]
        bufb = rest[NBH:2 * NBH]
        sema = rest[2 * NBH:3 * NBH]
        semb = rest[3 * NBH:]
        c = lax.axis_index("c")
        s = lax.axis_index("s")
        wid = c * NS + s
        w0 = wid * n_per

        pltpu.sync_copy(src_hbm.at[pl.ds(w0, n_per)], sidx_v)
        pltpu.sync_copy(dst_hbm.at[pl.ds(w0, n_per)], didx_v)

        def _gather(k, b):
            pltpu.async_copy(h_hbm.at[sidx_v.at[k]], bufa[b], sema[b])
            pltpu.async_copy(hw_hbm.at[didx_v.at[k]], bufb[b], semb[b])

        _gather(0, 0)
        lane = lax.iota(jnp.int32, L)

        def _pair(q, _):
            for b in range(NBH):
                k = q * NBH + b
                nb = (b + 1) % NBH

                @pl.when(k + 1 < n_per)
                def _():
                    _gather(k + 1, nb)

                pltpu.make_async_copy(h_hbm.at[sidx_v.at[k]], bufa[b],
                                      sema[b]).wait()
                pltpu.make_async_copy(hw_hbm.at[didx_v.at[k]], bufb[b],
                                      semb[b]).wait()
                ba, bb = bufa[b], bufb[b]

                def _grp(g, _):
                    vec = jnp.zeros((L,), jnp.float32)
                    for i in range(L):
                        e = g * L + i
                        acc = ba[e, pl.ds(0, L)] * bb[e, pl.ds(0, L)]
                        for j in range(1, HJ):
                            acc = acc + (ba[e, pl.ds(j * L, L)]
                                         * bb[e, pl.ds(j * L, L)])
                        vec = jnp.where(lane == i, jnp.sum(acc), vec)
                    pr_v[k, pl.ds(g * L, L)] = vec
                    return 0
                lax.fori_loop(0, C // L, _grp, 0)
            return 0
        lax.fori_loop(0, n_per // NBH, _pair, 0)

        pltpu.sync_copy(pr_v, pred_hbm.at[pl.ds(w0, n_per)])

    return sc_hint


def _make_tc_encode(N, D, H, R):
    row = pl.BlockSpec((R, D), lambda i: (i, 0))
    full = pl.BlockSpec((D, H), lambda i: (0, 0))
    vec = pl.BlockSpec((1, H), lambda i: (0, 0))
    out_row = pl.BlockSpec((R, H), lambda i: (i, 0))

    def body(x_ref, wenc_ref, wmsg_ref, b_ref, xe_ref, p_ref):
        xe = jnp.dot(x_ref[...], wenc_ref[...],
                     preferred_element_type=jnp.float32,
                     precision=lax.Precision.HIGHEST)
        xe_ref[...] = xe
        p_ref[...] = jnp.dot(xe, wmsg_ref[...],
                             preferred_element_type=jnp.float32,
                             precision=lax.Precision.HIGHEST) + b_ref[...]

    return pl.pallas_call(
        body,
        grid=(N // R,),
        in_specs=[row, full, full, vec],
        out_specs=[out_row, out_row],
        out_shape=[jax.ShapeDtypeStruct((N, H), jnp.float32)] * 2,
    )


def _make_tc_update(N, N_pad, H, R, with_p):
    row = pl.BlockSpec((R, H), lambda i: (i, 0))
    agg_spec = pl.BlockSpec((NC, R, H), lambda i: (0, i, 0))
    full = pl.BlockSpec((H, H), lambda i: (0, 0))
    vec = pl.BlockSpec((1, H), lambda i: (0, 0))

    def body(agg_ref, h_ref, xe_ref, wagg_ref, wself_ref, wmsg_ref, b_ref,
             whint_ref, hn_ref, hb_ref, hwb_ref, *p_refs):
        agg = agg_ref[0] + agg_ref[1]
        hn = jnp.dot(agg, wagg_ref[...], preferred_element_type=jnp.float32,
                     precision=lax.Precision.HIGHEST)
        hn = hn + jnp.dot(h_ref[...], wself_ref[...],
                          preferred_element_type=jnp.float32,
                          precision=lax.Precision.HIGHEST)
        hn = jnp.maximum(hn, 0.0)
        hn_ref[...] = hn
        hb_ref[...] = hn
        hwb_ref[...] = hn * whint_ref[...]
        if with_p:
            p_refs[0][...] = (
                jnp.dot(hn + xe_ref[...], wmsg_ref[...],
                        preferred_element_type=jnp.float32,
                        precision=lax.Precision.HIGHEST) + b_ref[...])

    n_out = 4 if with_p else 3
    out_shape = [jax.ShapeDtypeStruct((N, H), jnp.float32)] * n_out
    return pl.pallas_call(
        body,
        grid=(N // R,),
        in_specs=[agg_spec, row, row, full, full, full, vec, vec],
        out_specs=[row] * n_out,
        out_shape=out_shape,
    )


def _make_tc_out(N, H, R):
    row = pl.BlockSpec((R, H), lambda i: (i, 0))
    vec = pl.BlockSpec((1, H), lambda i: (0, 0))
    out_spec = pl.BlockSpec((R, 1), lambda i: (i, 0))

    def body(h_ref, wout_ref, o_ref):
        o_ref[...] = jnp.sum(h_ref[...] * wout_ref[...], axis=1, keepdims=True)

    return pl.pallas_call(
        body,
        grid=(N // R,),
        in_specs=[row, vec],
        out_specs=out_spec,
        out_shape=jax.ShapeDtypeStruct((N, 1), jnp.float32),
    )


def kernel(x, edge_index, trace_h, W_enc, W_msg, b_msg, w_eh, W_agg, W_self,
           w_hint, W_out):
    N, D = x.shape
    H = W_msg.shape[0]
    T, E = trace_h.shape
    R = 1000
    N_pad = -(-N // (NS * 128)) * (NS * 128)
    n_per = -(-E // (NW * C * 2 * G)) * (2 * G)
    E_pad = NW * n_per * C
    assert N % R == 0

    src = edge_index[0]
    dst = edge_index[1]
    pad = E_pad - E
    src_p = jnp.pad(src, (0, pad)).reshape(-1, C)
    pad_dst = N + jnp.arange(pad, dtype=dst.dtype) % (N_pad - N)
    dstm_p = jnp.concatenate([dst, pad_dst]).reshape(-1, C)
    dsth_p = jnp.pad(dst, (0, pad)).reshape(-1, C)
    tr_p = jnp.pad(trace_h, ((0, 0), (0, pad))).reshape(T, -1, C)
    b2 = b_msg.reshape(1, H)
    whint2 = w_hint.reshape(1, H)

    sc_msg = _make_sc_msg(N_pad, n_per, H)
    sc_hint = _make_sc_hint(N, n_per, H)
    tc_encode = _make_tc_encode(N, D, H, R)
    tc_update = _make_tc_update(N, N_pad, H, R, True)
    tc_update_last = _make_tc_update(N, N_pad, H, R, False)
    tc_out = _make_tc_out(N, H, R)

    xe, p = tc_encode(x, W_enc, W_msg, b2)
    h = jnp.zeros((N, H), dtype=x.dtype)
    preds = []
    for t in range(T):
        agg = sc_msg(p, src_p, dstm_p, tr_p[t], w_eh)
        if t + 1 < T:
            h, hb, hwb, p = tc_update(agg, h, xe, W_agg, W_self, W_msg, b2,
                                      whint2)
        else:
            h, hb, hwb = tc_update_last(agg, h, xe, W_agg, W_self, W_msg, b2,
                                        whint2)
        preds.append(sc_hint(hb, hwb, src_p, dsth_p).reshape(-1)[:E])
    out = tc_out(h, W_out.reshape(1, H))
    return out, jnp.stack(preds, axis=0)


# speedup vs baseline: 1.1870x; 1.1674x over previous
import functools

import jax
import jax.numpy as jnp
from jax import lax
from jax.experimental import pallas as pl
from jax.experimental.pallas import tpu as pltpu
from jax.experimental.pallas import tpu_sc as plsc

NC, NS = 2, 16
NW = NC * NS
L = 16
C = 128


def _make_sc_msg(N_pad, E, H):
    n_chunks = E // C
    n_base, n_extra = n_chunks // NW, n_chunks % NW
    rows_per = N_pad // NS
    zr = 128
    n_zero = rows_per // zr
    HJ = H // L

    mesh = plsc.VectorSubcoreMesh(core_axis_name="c", subcore_axis_name="s")

    @functools.partial(
        pl.kernel,
        out_type=jax.ShapeDtypeStruct((NC, N_pad, H), jnp.float32),
        mesh=mesh,
        compiler_params=pltpu.CompilerParams(needs_layout_passes=False),
        scratch_types=[
            pltpu.VMEM((H,), jnp.float32),
            pltpu.VMEM((C,), jnp.int32),
            pltpu.VMEM((C,), jnp.int32),
            pltpu.VMEM((C,), jnp.float32),
            pltpu.VMEM((C, H), jnp.float32),
            pltpu.VMEM((zr, H), jnp.float32),
            pltpu.VMEM_SHARED((N_pad, H), jnp.float32),
            pltpu.SemaphoreType.DMA,
        ],
    )
    def sc_msg(p_hbm, src_hbm, dst_hbm, tr_hbm, weh_hbm, agg_hbm,
               weh_v, sidx_v, didx_v, tr_v, buf_v, zero_v, agg_sh, sem):
        c = lax.axis_index("c")
        s = lax.axis_index("s")
        wid = c * NS + s

        pltpu.sync_copy(weh_hbm, weh_v)

        def _zrow(i, _):
            for j in range(HJ):
                zero_v[i, pl.ds(j * L, L)] = jnp.zeros((L,), jnp.float32)
            return 0
        lax.fori_loop(0, zr, _zrow, 0)
        for k in range(n_zero):
            pltpu.sync_copy(zero_v, agg_sh.at[pl.ds(s * rows_per + k * zr, zr)])
        plsc.subcore_barrier()

        weh = [weh_v[pl.ds(j * L, L)] for j in range(HJ)]

        def _chunk(i, _):
            base = (i * NW + wid) * C
            pltpu.sync_copy(src_hbm.at[pl.ds(base, C)], sidx_v)
            pltpu.sync_copy(dst_hbm.at[pl.ds(base, C)], didx_v)
            pltpu.sync_copy(tr_hbm.at[pl.ds(base, C)], tr_v)
            pltpu.async_copy(p_hbm.at[sidx_v], buf_v, sem).wait()

            def _grp(g, _):
                trv = tr_v[pl.ds(g * L, L)]
                for i2 in range(L):
                    t = trv[i2]
                    e = g * L + i2
                    for j in range(HJ):
                        v = buf_v[e, pl.ds(j * L, L)]
                        buf_v[e, pl.ds(j * L, L)] = jnp.maximum(
                            v + t * weh[j], 0.0)
                return 0
            lax.fori_loop(0, C // L, _grp, 0)

            pltpu.sync_copy(buf_v, agg_sh.at[didx_v], add=True)
            return 0

        n_i = n_base + (wid < n_extra).astype(jnp.int32)
        lax.fori_loop(0, n_i, _chunk, 0)
        plsc.subcore_barrier()

        pltpu.sync_copy(agg_sh.at[pl.ds(s * rows_per, rows_per)],
                        agg_hbm.at[c, pl.ds(s * rows_per, rows_per)])

    return sc_msg


def _make_sc_hint(N, E, H):
    n_chunks = E // C
    n_base, n_extra = n_chunks // NW, n_chunks % NW
    HJ = H // L

    mesh = plsc.VectorSubcoreMesh(core_axis_name="c", subcore_axis_name="s")

    @functools.partial(
        pl.kernel,
        out_type=jax.ShapeDtypeStruct((E,), jnp.float32),
        mesh=mesh,
        compiler_params=pltpu.CompilerParams(needs_layout_passes=False),
        scratch_types=[
            pltpu.VMEM((C,), jnp.int32),
            pltpu.VMEM((C,), jnp.int32),
            pltpu.VMEM((C, H), jnp.float32),
            pltpu.VMEM((C, H), jnp.float32),
            pltpu.VMEM((C,), jnp.float32),
            pltpu.SemaphoreType.DMA,
            pltpu.SemaphoreType.DMA,
        ],
    )
    def sc_hint(h_hbm, hw_hbm, src_hbm, dst_hbm, pred_hbm,
                sidx_v, didx_v, bufa_v, bufb_v, pr_v, sema, semb):
        c = lax.axis_index("c")
        s = lax.axis_index("s")
        wid = c * NS + s

        def _chunk(i, _):
            base = (i * NW + wid) * C
            pltpu.sync_copy(src_hbm.at[pl.ds(base, C)], sidx_v)
            pltpu.sync_copy(dst_hbm.at[pl.ds(base, C)], didx_v)
            cpa = pltpu.async_copy(h_hbm.at[sidx_v], bufa_v, sema)
            cpb = pltpu.async_copy(hw_hbm.at[didx_v], bufb_v, semb)
            cpa.wait()
            cpb.wait()

            lane = lax.iota(jnp.int32, L)

            def _grp(g, _):
                vec = jnp.zeros((L,), jnp.float32)
                for i2 in range(L):
                    e = g * L + i2
                    acc = bufa_v[e, pl.ds(0, L)] * bufb_v[e, pl.ds(0, L)]
                    for j in range(1, HJ):
                        acc = acc + (bufa_v[e, pl.ds(j * L, L)]
                                     * bufb_v[e, pl.ds(j * L, L)])
                    vec = jnp.where(lane == i2, jnp.sum(acc), vec)
                pr_v[pl.ds(g * L, L)] = vec
                return 0
            lax.fori_loop(0, C // L, _grp, 0)

            pltpu.sync_copy(pr_v, pred_hbm.at[pl.ds(base, C)])
            return 0

        n_i = n_base + (wid < n_extra).astype(jnp.int32)
        lax.fori_loop(0, n_i, _chunk, 0)

    return sc_hint


def _make_tc_encode(N, D, H, R):
    row = pl.BlockSpec((R, D), lambda i: (i, 0))
    full = pl.BlockSpec((D, H), lambda i: (0, 0))
    vec = pl.BlockSpec((1, H), lambda i: (0, 0))
    out_row = pl.BlockSpec((R, H), lambda i: (i, 0))

    def body(x_ref, wenc_ref, wmsg_ref, b_ref, xe_ref, p_ref):
        xe = jnp.dot(x_ref[...], wenc_ref[...],
                     preferred_element_type=jnp.float32,
                     precision=lax.Precision.HIGHEST)
        xe_ref[...] = xe
        p_ref[...] = jnp.dot(xe, wmsg_ref[...],
                             preferred_element_type=jnp.float32,
                             precision=lax.Precision.HIGHEST) + b_ref[...]

    return pl.pallas_call(
        body,
        grid=(N // R,),
        in_specs=[row, full, full, vec],
        out_specs=[out_row, out_row],
        out_shape=[jax.ShapeDtypeStruct((N, H), jnp.float32)] * 2,
    )


def _make_tc_update(N, N_pad, H, R, with_p):
    row = pl.BlockSpec((R, H), lambda i: (i, 0))
    agg_spec = pl.BlockSpec((NC, R, H), lambda i: (0, i, 0))
    full = pl.BlockSpec((H, H), lambda i: (0, 0))
    vec = pl.BlockSpec((1, H), lambda i: (0, 0))

    def body(agg_ref, h_ref, xe_ref, wagg_ref, wself_ref, wmsg_ref, b_ref,
             whint_ref, hn_ref, hw_ref, *p_refs):
        agg = agg_ref[0] + agg_ref[1]
        hn = jnp.dot(agg, wagg_ref[...], preferred_element_type=jnp.float32,
                     precision=lax.Precision.HIGHEST)
        hn = hn + jnp.dot(h_ref[...], wself_ref[...],
                          preferred_element_type=jnp.float32,
                          precision=lax.Precision.HIGHEST)
        hn = jnp.maximum(hn, 0.0)
        hn_ref[...] = hn
        hw_ref[...] = hn * whint_ref[...]
        if with_p:
            p_refs[0][...] = (
                jnp.dot(hn + xe_ref[...], wmsg_ref[...],
                        preferred_element_type=jnp.float32,
                        precision=lax.Precision.HIGHEST) + b_ref[...])

    n_out = 3 if with_p else 2
    return pl.pallas_call(
        body,
        grid=(N // R,),
        in_specs=[agg_spec, row, row, full, full, full, vec, vec],
        out_specs=[row] * n_out,
        out_shape=[jax.ShapeDtypeStruct((N, H), jnp.float32)] * n_out,
    )


def _make_tc_out(N, H, R):
    row = pl.BlockSpec((R, H), lambda i: (i, 0))
    vec = pl.BlockSpec((1, H), lambda i: (0, 0))
    out_spec = pl.BlockSpec((R, 1), lambda i: (i, 0))

    def body(h_ref, wout_ref, o_ref):
        o_ref[...] = jnp.sum(h_ref[...] * wout_ref[...], axis=1, keepdims=True)

    return pl.pallas_call(
        body,
        grid=(N // R,),
        in_specs=[row, vec],
        out_specs=out_spec,
        out_shape=jax.ShapeDtypeStruct((N, 1), jnp.float32),
    )


def kernel(x, edge_index, trace_h, W_enc, W_msg, b_msg, w_eh, W_agg, W_self,
           w_hint, W_out):
    N, D = x.shape
    H = W_msg.shape[0]
    T, E = trace_h.shape
    R = 1000
    N_pad = -(-N // (NS * 128)) * (NS * 128)
    assert N % R == 0 and E % C == 0

    src = edge_index[0]
    dst = edge_index[1]
    b2 = b_msg.reshape(1, H)
    whint2 = w_hint.reshape(1, H)

    sc_msg = _make_sc_msg(N_pad, E, H)
    sc_hint = _make_sc_hint(N, E, H)
    tc_encode = _make_tc_encode(N, D, H, R)
    tc_update = _make_tc_update(N, N_pad, H, R, True)
    tc_update_last = _make_tc_update(N, N_pad, H, R, False)
    tc_out = _make_tc_out(N, H, R)

    xe, p = tc_encode(x, W_enc, W_msg, b2)
    h = jnp.zeros((N, H), dtype=x.dtype)
    preds = []
    for t in range(T):
        agg = sc_msg(p, src, dst, trace_h[t], w_eh)
        if t + 1 < T:
            h, hw, p = tc_update(agg, h, xe, W_agg, W_self, W_msg, b2, whint2)
        else:
            h, hw = tc_update_last(agg, h, xe, W_agg, W_self, W_msg, b2,
                                   whint2)
        preds.append(sc_hint(h, hw, src, dst))
    out = tc_out(h, W_out.reshape(1, H))
    return out, jnp.stack(preds, axis=0)


# speedup vs baseline: 1.2826x; 1.0805x over previous
import functools

import jax
import jax.numpy as jnp
from jax import lax
from jax.experimental import pallas as pl
from jax.experimental.pallas import tpu as pltpu
from jax.experimental.pallas import tpu_sc as plsc

NC, NS = 2, 16
NW = NC * NS
L = 16
C = 128


def _make_sc_msg(N_pad, E, H):
    n_chunks = E // C
    n_base, n_extra = n_chunks // NW, n_chunks % NW
    rows_per = N_pad // NS
    zr = 128
    n_zero = rows_per // zr
    HJ = H // L

    mesh = plsc.VectorSubcoreMesh(core_axis_name="c", subcore_axis_name="s")

    @functools.partial(
        pl.kernel,
        out_type=jax.ShapeDtypeStruct((NC, N_pad, H), jnp.float32),
        mesh=mesh,
        compiler_params=pltpu.CompilerParams(needs_layout_passes=False),
        scratch_types=[
            pltpu.VMEM((H,), jnp.float32),
            pltpu.VMEM((C,), jnp.int32),
            pltpu.VMEM((C,), jnp.int32),
            pltpu.VMEM((C,), jnp.float32),
            pltpu.VMEM((C, H), jnp.float32),
            pltpu.VMEM((zr, H), jnp.float32),
            pltpu.VMEM_SHARED((N_pad, H), jnp.float32),
            pltpu.SemaphoreType.DMA,
        ],
    )
    def sc_msg(p_hbm, src_hbm, dst_hbm, tr_hbm, weh_hbm, agg_hbm,
               weh_v, sidx_v, didx_v, tr_v, buf_v, zero_v, agg_sh, sem):
        c = lax.axis_index("c")
        s = lax.axis_index("s")
        wid = c * NS + s

        pltpu.sync_copy(weh_hbm, weh_v)

        def _zrow(i, _):
            for j in range(HJ):
                zero_v[i, pl.ds(j * L, L)] = jnp.zeros((L,), jnp.float32)
            return 0
        lax.fori_loop(0, zr, _zrow, 0)
        for k in range(n_zero):
            pltpu.sync_copy(zero_v, agg_sh.at[pl.ds(s * rows_per + k * zr, zr)])
        plsc.subcore_barrier()

        weh = [weh_v[pl.ds(j * L, L)] for j in range(HJ)]

        def _chunk(i, _):
            base = (i * NW + wid) * C
            pltpu.sync_copy(src_hbm.at[pl.ds(base, C)], sidx_v)
            cp = pltpu.async_copy(p_hbm.at[sidx_v], buf_v, sem)
            pltpu.sync_copy(dst_hbm.at[pl.ds(base, C)], didx_v)
            pltpu.sync_copy(tr_hbm.at[pl.ds(base, C)], tr_v)
            cp.wait()

            def _grp(g, _):
                trv = tr_v[pl.ds(g * L, L)]
                for i2 in range(L):
                    t = trv[i2]
                    e = g * L + i2
                    for j in range(HJ):
                        v = buf_v[e, pl.ds(j * L, L)]
                        buf_v[e, pl.ds(j * L, L)] = jnp.maximum(
                            v + t * weh[j], 0.0)
                return 0
            lax.fori_loop(0, C // L, _grp, 0)

            pltpu.sync_copy(buf_v, agg_sh.at[didx_v], add=True)
            return 0

        n_i = n_base + (wid < n_extra).astype(jnp.int32)
        lax.fori_loop(0, n_i, _chunk, 0)
        plsc.subcore_barrier()

        pltpu.sync_copy(agg_sh.at[pl.ds(s * rows_per, rows_per)],
                        agg_hbm.at[c, pl.ds(s * rows_per, rows_per)])

    return sc_msg


def _make_sc_hint(N, E, H):
    n_chunks = E // C
    n_base, n_extra = n_chunks // NW, n_chunks % NW
    HJ = H // L

    mesh = plsc.VectorSubcoreMesh(core_axis_name="c", subcore_axis_name="s")

    @functools.partial(
        pl.kernel,
        out_type=jax.ShapeDtypeStruct((E,), jnp.float32),
        mesh=mesh,
        compiler_params=pltpu.CompilerParams(needs_layout_passes=False),
        scratch_types=[
            pltpu.VMEM((C,), jnp.int32),
            pltpu.VMEM((C,), jnp.int32),
            pltpu.VMEM((C, H), jnp.float32),
            pltpu.VMEM((C, H), jnp.float32),
            pltpu.VMEM((C,), jnp.float32),
            pltpu.SemaphoreType.DMA,
            pltpu.SemaphoreType.DMA,
        ],
    )
    def sc_hint(h_hbm, hw_hbm, src_hbm, dst_hbm, pred_hbm,
                sidx_v, didx_v, bufa_v, bufb_v, pr_v, sema, semb):
        c = lax.axis_index("c")
        s = lax.axis_index("s")
        wid = c * NS + s

        def _chunk(i, _):
            base = (i * NW + wid) * C
            pltpu.sync_copy(src_hbm.at[pl.ds(base, C)], sidx_v)
            pltpu.sync_copy(dst_hbm.at[pl.ds(base, C)], didx_v)
            cpa = pltpu.async_copy(h_hbm.at[sidx_v], bufa_v, sema)
            cpb = pltpu.async_copy(hw_hbm.at[didx_v], bufb_v, semb)
            cpa.wait()
            cpb.wait()

            lane = lax.iota(jnp.int32, L)

            def _grp(g, _):
                vec = jnp.zeros((L,), jnp.float32)
                for i2 in range(L):
                    e = g * L + i2
                    acc = bufa_v[e, pl.ds(0, L)] * bufb_v[e, pl.ds(0, L)]
                    for j in range(1, HJ):
                        acc = acc + (bufa_v[e, pl.ds(j * L, L)]
                                     * bufb_v[e, pl.ds(j * L, L)])
                    vec = jnp.where(lane == i2, jnp.sum(acc), vec)
                pr_v[pl.ds(g * L, L)] = vec
                return 0
            lax.fori_loop(0, C // L, _grp, 0)

            pltpu.sync_copy(pr_v, pred_hbm.at[pl.ds(base, C)])
            return 0

        n_i = n_base + (wid < n_extra).astype(jnp.int32)
        lax.fori_loop(0, n_i, _chunk, 0)

    return sc_hint


def _make_tc_encode(N, D, H, R):
    row = pl.BlockSpec((R, D), lambda i: (i, 0))
    full = pl.BlockSpec((D, H), lambda i: (0, 0))
    vec = pl.BlockSpec((1, H), lambda i: (0, 0))
    out_row = pl.BlockSpec((R, H), lambda i: (i, 0))

    def body(x_ref, wenc_ref, wmsg_ref, b_ref, xe_ref, p_ref):
        xe = jnp.dot(x_ref[...], wenc_ref[...],
                     preferred_element_type=jnp.float32,
                     precision=lax.Precision.HIGHEST)
        xe_ref[...] = xe
        p_ref[...] = jnp.dot(xe, wmsg_ref[...],
                             preferred_element_type=jnp.float32,
                             precision=lax.Precision.HIGHEST) + b_ref[...]

    return pl.pallas_call(
        body,
        grid=(N // R,),
        in_specs=[row, full, full, vec],
        out_specs=[out_row, out_row],
        out_shape=[jax.ShapeDtypeStruct((N, H), jnp.float32)] * 2,
    )


def _make_tc_update(N, N_pad, H, R, with_p):
    row = pl.BlockSpec((R, H), lambda i: (i, 0))
    agg_spec = pl.BlockSpec((NC, R, H), lambda i: (0, i, 0))
    full = pl.BlockSpec((H, H), lambda i: (0, 0))
    vec = pl.BlockSpec((1, H), lambda i: (0, 0))

    def body(agg_ref, h_ref, xe_ref, wagg_ref, wself_ref, wmsg_ref, b_ref,
             whint_ref, hn_ref, hw_ref, *p_refs):
        agg = agg_ref[0] + agg_ref[1]
        hn = jnp.dot(agg, wagg_ref[...], preferred_element_type=jnp.float32,
                     precision=lax.Precision.HIGHEST)
        hn = hn + jnp.dot(h_ref[...], wself_ref[...],
                          preferred_element_type=jnp.float32,
                          precision=lax.Precision.HIGHEST)
        hn = jnp.maximum(hn, 0.0)
        hn_ref[...] = hn
        hw_ref[...] = hn * whint_ref[...]
        if with_p:
            p_refs[0][...] = (
                jnp.dot(hn + xe_ref[...], wmsg_ref[...],
                        preferred_element_type=jnp.float32,
                        precision=lax.Precision.HIGHEST) + b_ref[...])

    n_out = 3 if with_p else 2
    return pl.pallas_call(
        body,
        grid=(N // R,),
        in_specs=[agg_spec, row, row, full, full, full, vec, vec],
        out_specs=[row] * n_out,
        out_shape=[jax.ShapeDtypeStruct((N, H), jnp.float32)] * n_out,
    )


def _make_tc_out(N, H, R):
    row = pl.BlockSpec((R, H), lambda i: (i, 0))
    vec = pl.BlockSpec((1, H), lambda i: (0, 0))
    out_spec = pl.BlockSpec((R, 1), lambda i: (i, 0))

    def body(h_ref, wout_ref, o_ref):
        o_ref[...] = jnp.sum(h_ref[...] * wout_ref[...], axis=1, keepdims=True)

    return pl.pallas_call(
        body,
        grid=(N // R,),
        in_specs=[row, vec],
        out_specs=out_spec,
        out_shape=jax.ShapeDtypeStruct((N, 1), jnp.float32),
    )


def kernel(x, edge_index, trace_h, W_enc, W_msg, b_msg, w_eh, W_agg, W_self,
           w_hint, W_out):
    N, D = x.shape
    H = W_msg.shape[0]
    T, E = trace_h.shape
    R = 1000
    N_pad = -(-N // (NS * 128)) * (NS * 128)
    assert N % R == 0 and E % C == 0

    src = edge_index[0]
    dst = edge_index[1]
    b2 = b_msg.reshape(1, H)
    whint2 = w_hint.reshape(1, H)

    sc_msg = _make_sc_msg(N_pad, E, H)
    sc_hint = _make_sc_hint(N, E, H)
    tc_encode = _make_tc_encode(N, D, H, R)
    tc_update = _make_tc_update(N, N_pad, H, R, True)
    tc_update_last = _make_tc_update(N, N_pad, H, R, False)
    tc_out = _make_tc_out(N, H, R)

    xe, p = tc_encode(x, W_enc, W_msg, b2)
    h = jnp.zeros((N, H), dtype=x.dtype)
    preds = []
    for t in range(T):
        agg = sc_msg(p, src, dst, trace_h[t], w_eh)
        if t + 1 < T:
            h, hw, p = tc_update(agg, h, xe, W_agg, W_self, W_msg, b2, whint2)
        else:
            h, hw = tc_update_last(agg, h, xe, W_agg, W_self, W_msg, b2,
                                   whint2)
        preds.append(sc_hint(h, hw, src, dst))
    out = tc_out(h, W_out.reshape(1, H))
    return out, jnp.stack(preds, axis=0)
